# baseline (speedup 1.0000x reference)
import functools

import jax
import jax.numpy as jnp
from jax import lax
from jax.experimental import pallas as pl
from jax.experimental.pallas import tpu as pltpu
from jax.experimental.pallas import tpu_sc as plsc

N = 10000
E = 320000
IN_DIM = 128
HID_DIM = 256
OUT_DIM = 128

NC = 2
NS = 16
K = 128
NB = 2
IDXG = 8
EPAD = 327680
ROWS_PAD = 10112
RPT_Z = 632
RPT_O = 624
D = 128
DEG_W = 16

_mesh = plsc.VectorSubcoreMesh(core_axis_name="c", subcore_axis_name="s")


def _zero_fill(buf, nrows, width):
    def z(i, _):
        for j in range(width // 16):
            buf[i, pl.ds(j * 16, 16)] = jnp.zeros((16,), jnp.float32)
        return 0
    lax.fori_loop(0, nrows, z, 0)


def _zero_accum(zeros_hbm, accum, s):
    r0 = pl.multiple_of(s * RPT_Z, 8)
    pltpu.sync_copy(zeros_hbm, accum.at[pl.ds(r0, RPT_Z)])


def _copy_out(accum, out, s):
    q0 = pl.multiple_of(s * RPT_O, 8)
    pltpu.sync_copy(accum.at[pl.ds(q0, RPT_O)], out.at[pl.ds(q0, RPT_O)])

    @pl.when(s == NS - 1)
    def _():
        tail = N - NS * RPT_O
        pltpu.sync_copy(accum.at[pl.ds(NS * RPT_O, tail)],
                        out.at[pl.ds(NS * RPT_O, tail)])


def _deg_body(dst_hbm, z_hbm, out0, out1, buf, idxb, accum):
    c = lax.axis_index("c")
    s = lax.axis_index("s")
    _zero_accum(z_hbm, accum, s)

    def ones(i, _):
        buf[i, :] = jnp.ones((DEG_W,), jnp.float32)
        return 0
    lax.fori_loop(0, K, ones, 0)
    plsc.subcore_barrier()

    nch = EPAD // K // (NC * NS)
    base_ch = (s * NC + c) * nch

    def it(g, _):
        ch0 = pl.multiple_of(base_ch + g * IDXG, IDXG)
        pltpu.sync_copy(dst_hbm.at[pl.ds(ch0, IDXG)], idxb)
        for k in range(IDXG):
            pltpu.sync_copy(buf, accum.at[idxb.at[k]], add=True)
        return 0
    lax.fori_loop(0, nch // IDXG, it, 0)
    plsc.subcore_barrier()

    @pl.when(c == 0)
    def _():
        _copy_out(accum, out0, s)

    @pl.when(c == 1)
    def _():
        _copy_out(accum, out1, s)


_deg_kernel = functools.partial(
    pl.kernel,
    mesh=_mesh,
    out_type=[jax.ShapeDtypeStruct((N, DEG_W), jnp.float32)] * 2,
    scratch_types=[
        pltpu.VMEM((K, DEG_W), jnp.float32),
        pltpu.VMEM((IDXG, K), jnp.int32),
        pltpu.VMEM_SHARED((ROWS_PAD, DEG_W), jnp.float32),
    ],
)(_deg_body)


def _edge_body(split_edges, src_hbm, dst_hbm, z_hbm, t0, t1, o0, o1,
               srcb, dstb, rows, accum, gs0, gs1, ss0, ss1):
    gsems = (gs0, gs1)
    ssems = (ss0, ss1)
    c = lax.axis_index("c")
    s = lax.axis_index("s")
    _zero_accum(z_hbm, accum, s)
    plsc.subcore_barrier()

    nch = EPAD // K // (NC * NS if split_edges else NS)
    base_ch = ((s * NC + c) if split_edges else s) * nch

    def run(tbl, out):
        def it(g, _):
            ch0 = pl.multiple_of(base_ch + g * IDXG, IDXG)
            pltpu.sync_copy(src_hbm.at[pl.ds(ch0, IDXG)], srcb)
            pltpu.sync_copy(dst_hbm.at[pl.ds(ch0, IDXG)], dstb)
            hg = [None, None]
            hs = [None, None]
            hg[0] = pltpu.async_copy(tbl.at[srcb.at[0]], rows.at[0],
                                     gsems[0])
            for k in range(IDXG):
                b = k % 2
                hg[b].wait()
                hs[b] = pltpu.async_copy(rows.at[b], accum.at[dstb.at[k]],
                                         ssems[b], add=True)
                if k + 1 < IDXG:
                    nb = 1 - b
                    if k >= 1:
                        hs[nb].wait()
                    hg[nb] = pltpu.async_copy(tbl.at[srcb.at[k + 1]],
                                              rows.at[nb], gsems[nb])
            hs[(IDXG - 1) % 2].wait()
            return 0
        lax.fori_loop(0, nch // IDXG, it, 0)
        plsc.subcore_barrier()
        _copy_out(accum, out, s)

    @pl.when(c == 0)
    def _():
        run(t0, o0)

    @pl.when(c == 1)
    def _():
        run(t1, o1)


def _make_edge_kernel(split_edges):
    return functools.partial(
        pl.kernel,
        mesh=_mesh,
        out_type=[jax.ShapeDtypeStruct((N, D), jnp.float32)] * 2,
        scratch_types=[
            pltpu.VMEM((IDXG, K), jnp.int32),
            pltpu.VMEM((IDXG, K), jnp.int32),
            pltpu.VMEM((NB, K, D), jnp.float32),
            pltpu.VMEM_SHARED((ROWS_PAD, D), jnp.float32),
        ] + [pltpu.SemaphoreType.DMA] * (2 * NB),
    )(functools.partial(_edge_body, split_edges))


_edge_l1 = _make_edge_kernel(False)
_edge_l2 = _make_edge_kernel(True)


BM = 1000


def _mm1_body(x_ref, d_ref, w_ref, o1_ref, o2_ref):
    xa = x_ref[...] * d_ref[...]
    h = jnp.dot(xa, w_ref[...], preferred_element_type=jnp.float32)
    o1_ref[...] = h[:, :128]
    o2_ref[...] = h[:, 128:]


def _mm2_body(alo_ref, ahi_ref, glo_ref, ghi_ref, d_ref, b1_ref, w2_ref, o_ref):
    d = d_ref[...]
    b = b1_ref[...]
    ulo = d * jnp.maximum(d * (alo_ref[...] + glo_ref[...]) + b[:, :128], 0.0)
    uhi = d * jnp.maximum(d * (ahi_ref[...] + ghi_ref[...]) + b[:, 128:], 0.0)
    w = w2_ref[...]
    o_ref[...] = (jnp.dot(ulo, w[:128, :], preferred_element_type=jnp.float32)
                  + jnp.dot(uhi, w[128:, :], preferred_element_type=jnp.float32))


def _ew_body(a0_ref, a1_ref, g_ref, d_ref, b2_ref, o_ref):
    o_ref[...] = (d_ref[...] * (a0_ref[...] + a1_ref[...] + g_ref[...])
                  + b2_ref[...])


def _row_spec(w):
    return pl.BlockSpec((BM, w), lambda i: (i, 0))


def _full_spec(h, w):
    return pl.BlockSpec((h, w), lambda i: (0, 0))


def kernel(x, edge_index, W1, b1, W2, b2):
    src = edge_index[0].astype(jnp.int32)
    dst = edge_index[1].astype(jnp.int32)
    pad = EPAD - E
    srcp = jnp.concatenate([src, jnp.zeros((pad,), jnp.int32)])
    dstp = jnp.concatenate([dst, jnp.full((pad,), N, jnp.int32)])
    srcp = srcp.reshape(EPAD // K, K)
    dstp = dstp.reshape(EPAD // K, K)

    zd = jnp.zeros((RPT_Z, DEG_W), jnp.float32)
    ze = jnp.zeros((RPT_Z, D), jnp.float32)
    c0, c1 = _deg_kernel(dstp, zd)
    cnt = (c0 + c1)[:, 0]
    dinv = lax.rsqrt(cnt + 1.0)
    dcol = dinv[:, None]

    g1lo, g1hi = pl.pallas_call(
        _mm1_body,
        grid=(N // BM,),
        in_specs=[_row_spec(128), _row_spec(1), _full_spec(128, 256)],
        out_specs=[_row_spec(128)] * 2,
        out_shape=[jax.ShapeDtypeStruct((N, 128), jnp.float32)] * 2,
    )(x, dcol, W1)

    a1lo, a1hi = _edge_l1(srcp, dstp, ze, g1lo, g1hi)

    g2 = pl.pallas_call(
        _mm2_body,
        grid=(N // BM,),
        in_specs=[_row_spec(128)] * 4 + [_row_spec(1), _full_spec(1, 256),
                                         _full_spec(256, 128)],
        out_specs=_row_spec(128),
        out_shape=jax.ShapeDtypeStruct((N, 128), jnp.float32),
    )(a1lo, a1hi, g1lo, g1hi, dcol, b1.reshape(1, HID_DIM), W2)

    a20, a21 = _edge_l2(srcp, dstp, ze, g2, g2)

    out = pl.pallas_call(
        _ew_body,
        grid=(N // BM,),
        in_specs=[_row_spec(128)] * 3 + [_row_spec(1), _full_spec(1, 128)],
        out_specs=_row_spec(128),
        out_shape=jax.ShapeDtypeStruct((N, OUT_DIM), jnp.float32),
    )(a20, a21, g2, dcol, b2.reshape(1, OUT_DIM))
    return out
